# baseline (speedup 1.0000x reference)
import functools

import jax
import jax.numpy as jnp
from jax.experimental import pallas as pl
from jax.experimental.pallas import tpu as pltpu

_N = 8400
_NP = 8448
_R = 66
_RP = 72
_C = 80
_K = 100
_SCORE_THR = 0.05
_IOU_THR = 0.65
_BIGI = 1 << 20


def _tpose(x):
    n = x.shape[1]
    eye = (jax.lax.broadcasted_iota(jnp.int32, (n, n), 0)
           == jax.lax.broadcasted_iota(jnp.int32, (n, n), 1)).astype(jnp.float32)
    return jax.lax.dot_general(eye, x, (((1,), (1,)), ((), ())),
                               precision=jax.lax.Precision.HIGHEST,
                               preferred_element_type=jnp.float32)


def _nms_body(cls_ref, bp_ref, pri_ref, obj_ref, dets_ref, lbl_ref, keep_ref,
              m_ref, lbl_scr):
    f32 = jnp.float32
    i32 = jnp.int32
    riota = jax.lax.broadcasted_iota(i32, (_R, 128), 0)
    liota = jax.lax.broadcasted_iota(i32, (_R, 128), 1)
    giota = jnp.where(riota == _R - 1, (_N - 128) + liota,
                      riota * 128 + liota)
    kiota = jax.lax.broadcasted_iota(i32, (1, 128), 1)

    ciota = jax.lax.broadcasted_iota(i32, (_C, 128), 0)
    for b in range(_R):
        start = 128 * b if b < _R - 1 else _N - 128
        tb = jnp.transpose(cls_ref[pl.ds(start, 128), :])
        mb = jnp.max(tb, axis=0, keepdims=True)
        lb = jnp.min(jnp.where(tb == mb, ciota, _BIGI), axis=0, keepdims=True)
        m_ref[b:b + 1, :] = mb
        lbl_scr[b:b + 1, :] = lb.astype(f32)
    m = m_ref[:]
    lblf = lbl_scr[:]

    score = jax.nn.sigmoid(m) * jax.nn.sigmoid(obj_ref[:])
    masked = jnp.where(score >= _SCORE_THR, score, -1.0)
    masked = jnp.where((riota == _R - 1) & (liota < 48), -2.0, masked)

    st = pri_ref[2]
    cx = bp_ref[0] * st + pri_ref[0]
    cy = bp_ref[1] * st + pri_ref[1]
    w2 = jnp.exp(bp_ref[2]) * st * 0.5
    h2 = jnp.exp(bp_ref[3]) * st * 0.5
    bx1 = cx - w2
    by1 = cy - h2
    bx2 = cx + w2
    by2 = cy + h2

    sl8 = jax.lax.broadcasted_iota(i32, (8, 128), 0)
    ones1 = jnp.ones((128, 1), f32)
    klf = kiota.astype(f32)

    def t_cond(s):
        _, _, _, written = s
        return written < _K

    def t_body(s):
        cur, idxf, scc, written = s
        C = jnp.max(cur, axis=0, keepdims=True)
        R = jnp.min(jnp.where(cur == C, riota, _BIGI), axis=0, keepdims=True)
        G = jnp.where(R == _R - 1, (_N - 128) + kiota,
                      R * 128 + kiota).astype(f32)
        cur2 = jnp.where(riota == R, -3.0, cur)
        m2v = jnp.max(cur2)
        r2 = jnp.min(jnp.where(cur2 == m2v, riota, _BIGI), axis=0,
                     keepdims=True)
        m2g = jnp.min(jnp.where(r2 == _R - 1, (_N - 128) + kiota,
                                r2 * 128 + kiota)).astype(f32)
        acc = (C > m2v) | ((C == m2v) & (G < m2g))
        mt = jnp.where(sl8 == 0, C, 0.0)
        mt = jnp.where(sl8 == 1, G, mt)
        tt = _tpose(mt)
        cc = tt[:, 0:1]
        gc = tt[:, 1:2]
        accc = (cc > m2v) | ((cc == m2v) & (gc < m2g))
        better = (acc & ((C > cc) | ((C == cc) & (G < gc)))).astype(f32)
        rank = jax.lax.dot_general(better, ones1, (((1,), (0,)), ((), ())),
                                   precision=jax.lax.Precision.HIGHEST,
                                   preferred_element_type=f32)
        slot = written.astype(f32) + rank
        ohs = accc.astype(f32) * (slot == klf).astype(f32)
        idxf = idxf + jax.lax.dot_general(
            ohs, gc, (((0,), (0,)), ((), ())),
            precision=jax.lax.Precision.HIGHEST, preferred_element_type=f32)
        scc = scc + jax.lax.dot_general(
            ohs, cc, (((0,), (0,)), ((), ())),
            precision=jax.lax.Precision.HIGHEST, preferred_element_type=f32)
        cur = jnp.where((riota == R) & acc, -3.0, cur)
        written = written + jnp.sum(acc.astype(f32)).astype(i32)
        return cur, idxf, scc, written

    zc = jnp.zeros((128, 1), f32)
    _, idxf, scc, _ = jax.lax.while_loop(
        t_cond, t_body, (masked, zc, zc, jnp.int32(0)))

    idxc = idxf.astype(i32)
    scol = scc
    last = idxc >= 128 * (_R - 1)
    rvec = jnp.where(last, _R - 1, idxc >> 7)
    lvec = jnp.where(last, idxc - (_N - 128), idxc & 127)
    ohr = (jax.lax.broadcasted_iota(i32, (128, _RP), 1) == rvec).astype(f32)
    ohl = (jax.lax.broadcasted_iota(i32, (128, 128), 1) == lvec).astype(f32)
    padrows = jnp.zeros((_RP - _R, 128), f32)
    ones1 = jnp.ones((128, 1), f32)

    def gather_col(ch):
        chp = jnp.concatenate([ch, padrows], axis=0)
        g = jax.lax.dot_general(ohr, chp, (((1,), (0,)), ((), ())),
                                precision=jax.lax.Precision.HIGHEST,
                                preferred_element_type=f32)
        return jax.lax.dot_general(g * ohl, ones1, (((1,), (0,)), ((), ())),
                                   precision=jax.lax.Precision.HIGHEST,
                                   preferred_element_type=f32)

    x1c = gather_col(bx1)
    y1c = gather_col(by1)
    x2c = gather_col(bx2)
    y2c = gather_col(by2)
    lbc = gather_col(lblf)

    C5 = jnp.concatenate([x1c, y1c, x2c, y2c, lbc,
                          jnp.zeros((128, 3), f32)], axis=1)
    T2 = _tpose(C5)
    x1r = T2[0:1, :]
    y1r = T2[1:2, :]
    x2r = T2[2:3, :]
    y2r = T2[3:4, :]
    lbr = T2[4:5, :]

    ix1 = jnp.maximum(x1c, x1r)
    iy1 = jnp.maximum(y1c, y1r)
    ix2 = jnp.minimum(x2c, x2r)
    iy2 = jnp.minimum(y2c, y2r)
    inter = jnp.maximum(ix2 - ix1, 0.0) * jnp.maximum(iy2 - iy1, 0.0)
    ar = (x2r - x1r) * (y2r - y1r)
    ac = (x2c - x1c) * (y2c - y1c)
    iou = inter / (ac + ar - inter + 1e-8)
    sup = ((iou >= _IOU_THR) & (lbc == lbr)).astype(f32)

    sl128 = jax.lax.broadcasted_iota(i32, (128, 128), 0)
    ll128 = jax.lax.broadcasted_iota(i32, (128, 128), 1)
    sup_u = sup * (sl128 < ll128).astype(f32)
    k0c = (scol > 0.0).astype(f32)

    def fix_cond(s):
        k, prev, it = s
        return jnp.logical_and(it < _K + 2, jnp.any(k != prev))

    def fix_body(s):
        k, prev, it = s
        t = jax.lax.dot_general(sup_u, k, (((0,), (0,)), ((), ())),
                                precision=jax.lax.Precision.HIGHEST,
                                preferred_element_type=f32)
        knew = k0c * (t == 0.0).astype(f32)
        return knew, k, it + 1

    keepc, _, _ = jax.lax.while_loop(
        fix_cond, fix_body, (k0c, k0c - 1.0, jnp.int32(0)))

    dets_ref[:, 0:1] = x1c[0:_K]
    dets_ref[:, 1:2] = y1c[0:_K]
    dets_ref[:, 2:3] = x2c[0:_K]
    dets_ref[:, 3:4] = y2c[0:_K]
    dets_ref[:, 4:5] = scol[0:_K]
    lbl_ref[:, :] = lbc
    keep_ref[:, :] = keepc


@jax.jit
def kernel(cls_scores, bbox_preds, objectness, priors):
    f32 = jnp.float32

    def overlap(x):
        return jnp.concatenate(
            [x[:, :128 * (_R - 1)].reshape(-1, _R - 1, 128),
             x[:, _N - 128:].reshape(-1, 1, 128)], axis=1)

    bpT = overlap(bbox_preds[0].T)
    priT = overlap(priors.T)
    objp = overlap(objectness[0][None, :])[0]

    dets, lblrow, keeprow = pl.pallas_call(
        _nms_body,
        out_shape=[
            jax.ShapeDtypeStruct((_K, 5), f32),
            jax.ShapeDtypeStruct((128, 1), f32),
            jax.ShapeDtypeStruct((128, 1), f32),
        ],
        scratch_shapes=[pltpu.VMEM((_R, 128), f32),
                        pltpu.VMEM((_R, 128), f32)],
    )(cls_scores[0], bpT, priT, objp)
    return (dets, lblrow[:_K, 0].astype(jnp.int32), keeprow[:_K, 0] != 0.0)


# speedup vs baseline: 1.3196x; 1.3196x over previous
import functools

import jax
import jax.numpy as jnp
from jax.experimental import pallas as pl
from jax.experimental.pallas import tpu as pltpu

_N = 8400
_NP = 8448
_R = 66
_RP = 72
_C = 80
_K = 100
_SCORE_THR = 0.05
_IOU_THR = 0.65
_BIGI = 1 << 20


def _tpose(x):
    n = x.shape[1]
    eye = (jax.lax.broadcasted_iota(jnp.int32, (n, n), 0)
           == jax.lax.broadcasted_iota(jnp.int32, (n, n), 1)).astype(jnp.float32)
    return jax.lax.dot_general(eye, x, (((1,), (1,)), ((), ())),
                               precision=jax.lax.Precision.HIGHEST,
                               preferred_element_type=jnp.float32)


def _nms_body(cls_ref, bp_ref, pri_ref, obj_ref, dets_ref, lbl_ref, keep_ref):
    f32 = jnp.float32
    i32 = jnp.int32
    riota = jax.lax.broadcasted_iota(i32, (_R, 128), 0)
    liota = jax.lax.broadcasted_iota(i32, (_R, 128), 1)
    giota = riota * 128 + liota
    kiota = jax.lax.broadcasted_iota(i32, (1, 128), 1)

    m = cls_ref[0]
    lblf = jnp.zeros((_R, 128), f32)
    for c in range(1, _C):
        x = cls_ref[c]
        gt = x > m
        m = jnp.where(gt, x, m)
        lblf = jnp.where(gt, f32(c), lblf)

    score = jax.nn.sigmoid(m) * jax.nn.sigmoid(obj_ref[:])
    masked = jnp.where(score >= _SCORE_THR, score, -1.0)
    masked = jnp.where(giota >= _N, -2.0, masked)

    st = pri_ref[2]
    cx = bp_ref[0] * st + pri_ref[0]
    cy = bp_ref[1] * st + pri_ref[1]
    w2 = jnp.exp(bp_ref[2]) * st * 0.5
    h2 = jnp.exp(bp_ref[3]) * st * 0.5
    bx1 = cx - w2
    by1 = cy - h2
    bx2 = cx + w2
    by2 = cy + h2

    sl8 = jax.lax.broadcasted_iota(i32, (8, 128), 0)
    ones1 = jnp.ones((128, 1), f32)
    klf = kiota.astype(f32)

    def t_cond(s):
        _, _, _, written = s
        return written < _K

    def t_body(s):
        cur, idxf, scc, written = s
        C = jnp.max(cur, axis=0, keepdims=True)
        R = jnp.min(jnp.where(cur == C, riota, _BIGI), axis=0, keepdims=True)
        G = (R * 128 + kiota).astype(f32)
        cur2 = jnp.where(riota == R, -3.0, cur)
        m2v = jnp.max(cur2)
        r2 = jnp.min(jnp.where(cur2 == m2v, riota, _BIGI), axis=0,
                     keepdims=True)
        m2g = jnp.min(r2 * 128 + kiota).astype(f32)
        acc = (C > m2v) | ((C == m2v) & (G < m2g))
        mt = jnp.where(sl8 == 0, C, 0.0)
        mt = jnp.where(sl8 == 1, G, mt)
        tt = _tpose(mt)
        cc = tt[:, 0:1]
        gc = tt[:, 1:2]
        accc = (cc > m2v) | ((cc == m2v) & (gc < m2g))
        better = (acc & ((C > cc) | ((C == cc) & (G < gc)))).astype(f32)
        rank = jax.lax.dot_general(better, ones1, (((1,), (0,)), ((), ())),
                                   precision=jax.lax.Precision.HIGHEST,
                                   preferred_element_type=f32)
        slot = written.astype(f32) + rank
        ohs = accc.astype(f32) * (slot == klf).astype(f32)
        idxf = idxf + jax.lax.dot_general(
            ohs, gc, (((0,), (0,)), ((), ())),
            precision=jax.lax.Precision.HIGHEST, preferred_element_type=f32)
        scc = scc + jax.lax.dot_general(
            ohs, cc, (((0,), (0,)), ((), ())),
            precision=jax.lax.Precision.HIGHEST, preferred_element_type=f32)
        cur = jnp.where((riota == R) & acc, -3.0, cur)
        written = written + jnp.sum(acc.astype(f32)).astype(i32)
        return cur, idxf, scc, written

    zc = jnp.zeros((128, 1), f32)
    _, idxf, scc, _ = jax.lax.while_loop(
        t_cond, t_body, (masked, zc, zc, jnp.int32(0)))

    idxc = idxf.astype(i32)
    scol = scc
    rvec = idxc >> 7
    lvec = idxc & 127
    ohr = (jax.lax.broadcasted_iota(i32, (128, _RP), 1) == rvec).astype(f32)
    ohl = (jax.lax.broadcasted_iota(i32, (128, 128), 1) == lvec).astype(f32)
    padrows = jnp.zeros((_RP - _R, 128), f32)
    ones1 = jnp.ones((128, 1), f32)

    def gather_col(ch):
        chp = jnp.concatenate([ch, padrows], axis=0)
        g = jax.lax.dot_general(ohr, chp, (((1,), (0,)), ((), ())),
                                precision=jax.lax.Precision.HIGHEST,
                                preferred_element_type=f32)
        return jax.lax.dot_general(g * ohl, ones1, (((1,), (0,)), ((), ())),
                                   precision=jax.lax.Precision.HIGHEST,
                                   preferred_element_type=f32)

    x1c = gather_col(bx1)
    y1c = gather_col(by1)
    x2c = gather_col(bx2)
    y2c = gather_col(by2)
    lbc = gather_col(lblf)

    C5 = jnp.concatenate([x1c, y1c, x2c, y2c, lbc,
                          jnp.zeros((128, 3), f32)], axis=1)
    T2 = _tpose(C5)
    x1r = T2[0:1, :]
    y1r = T2[1:2, :]
    x2r = T2[2:3, :]
    y2r = T2[3:4, :]
    lbr = T2[4:5, :]

    ix1 = jnp.maximum(x1c, x1r)
    iy1 = jnp.maximum(y1c, y1r)
    ix2 = jnp.minimum(x2c, x2r)
    iy2 = jnp.minimum(y2c, y2r)
    inter = jnp.maximum(ix2 - ix1, 0.0) * jnp.maximum(iy2 - iy1, 0.0)
    ar = (x2r - x1r) * (y2r - y1r)
    ac = (x2c - x1c) * (y2c - y1c)
    iou = inter / (ac + ar - inter + 1e-8)
    sup = ((iou >= _IOU_THR) & (lbc == lbr)).astype(f32)

    sl128 = jax.lax.broadcasted_iota(i32, (128, 128), 0)
    ll128 = jax.lax.broadcasted_iota(i32, (128, 128), 1)
    sup_u = sup * (sl128 < ll128).astype(f32)
    k0c = (scol > 0.0).astype(f32)

    def fix_cond(s):
        k, prev, it = s
        return jnp.logical_and(it < _K + 2, jnp.any(k != prev))

    def fix_body(s):
        k, prev, it = s
        t = jax.lax.dot_general(sup_u, k, (((0,), (0,)), ((), ())),
                                precision=jax.lax.Precision.HIGHEST,
                                preferred_element_type=f32)
        knew = k0c * (t == 0.0).astype(f32)
        return knew, k, it + 1

    keepc, _, _ = jax.lax.while_loop(
        fix_cond, fix_body, (k0c, k0c - 1.0, jnp.int32(0)))

    dets_ref[:, 0:1] = x1c[0:_K]
    dets_ref[:, 1:2] = y1c[0:_K]
    dets_ref[:, 2:3] = x2c[0:_K]
    dets_ref[:, 3:4] = y2c[0:_K]
    dets_ref[:, 4:5] = scol[0:_K]
    lbl_ref[:, :] = lbc
    keep_ref[:, :] = keepc


@jax.jit
def kernel(cls_scores, bbox_preds, objectness, priors):
    f32 = jnp.float32
    clsT = jnp.pad(cls_scores[0].T, ((0, 0), (0, _NP - _N)),
                   constant_values=-1e30).reshape(_C, _R, 128)
    bpT = jnp.pad(bbox_preds[0].T, ((0, 0), (0, _NP - _N))).reshape(4, _R, 128)
    priT = jnp.pad(priors.T, ((0, 0), (0, _NP - _N)),
                   constant_values=1.0).reshape(4, _R, 128)
    objp = jnp.pad(objectness[0], (0, _NP - _N),
                   constant_values=-100.0).reshape(_R, 128)

    dets, lblrow, keeprow = pl.pallas_call(
        _nms_body,
        out_shape=[
            jax.ShapeDtypeStruct((_K, 5), f32),
            jax.ShapeDtypeStruct((128, 1), f32),
            jax.ShapeDtypeStruct((128, 1), f32),
        ],
    )(clsT, bpT, priT, objp)
    return (dets, lblrow[:_K, 0].astype(jnp.int32), keeprow[:_K, 0] != 0.0)
